# baseline (device time: 21687 ns/iter reference)
import jax
import jax.numpy as jnp
from jax import lax
from jax.experimental import pallas as pl
from jax.experimental.pallas import tpu as pltpu

N_DEV = 8
B, SQ, SKV = 2, 256, 256
HQ_TOT, DH = 32, 64
H_LOC = HQ_TOT // N_DEV
CHUNK = H_LOC * DH
DM = 512
RH = SQ // 2

STREAMS = ((0, 0), (0, 1), (1, 0), (1, 1))
XORS = ((1, 3, 4), (4, 1, 3), (3, 4, 1), (1, 3, 4))


def kernel(x, Wq, K_ext, V_ext, Wo):
    my = lax.axis_index("i")
    Wq_loc = lax.dynamic_slice_in_dim(Wq, my * CHUNK, CHUNK, axis=1)
    Wo_loc = lax.dynamic_slice_in_dim(Wo, my * CHUNK, CHUNK, axis=0)
    K2 = K_ext.reshape(B, SKV, CHUNK)
    V2 = V_ext.reshape(B, SKV, CHUNK)

    def body(x_ref, wq_ref, k_ref, v_ref, wo_ref, out_ref,
             send_ref, recv_ref, send_sems, recv_sems):
        my_pos = lax.axis_index("i")

        barrier_sem = pltpu.get_barrier_semaphore()
        for d in (1, 3, 4):
            pl.semaphore_signal(barrier_sem, inc=1,
                                device_id=(jnp.bitwise_xor(my_pos, d),),
                                device_id_type=pl.DeviceIdType.MESH)
        pl.semaphore_wait(barrier_sem, 3)

        wq = (wq_ref[...] * 0.125).astype(jnp.bfloat16)
        wo = wo_ref[...].astype(jnp.bfloat16)

        def compute_partial(b, r):
            rows = slice(r * RH, (r + 1) * RH)
            qi = lax.broadcasted_iota(jnp.int32, (RH, SKV), 0) + r * RH
            ki = lax.broadcasted_iota(jnp.int32, (RH, SKV), 1)
            mask = (jnp.abs(qi - ki) <= 128) | (ki < 32) | (qi < 32)
            xb = x_ref[b, rows, :].astype(jnp.bfloat16)
            qb = jnp.dot(xb, wq, preferred_element_type=jnp.float32)
            ctx_parts = []
            for h in range(H_LOC):
                qh = qb[:, h * DH:(h + 1) * DH].astype(jnp.bfloat16)
                kh = k_ref[b, :, h * DH:(h + 1) * DH].astype(jnp.bfloat16)
                vh = v_ref[b, :, h * DH:(h + 1) * DH].astype(jnp.bfloat16)
                s = lax.dot_general(qh, kh, (((1,), (1,)), ((), ())),
                                    preferred_element_type=jnp.float32)
                w = jnp.where(mask, jnp.exp(s), 0.0)
                recip = 1.0 / jnp.sum(w, axis=-1, keepdims=True)
                ctx_parts.append(jnp.dot(w.astype(jnp.bfloat16), vh,
                                         preferred_element_type=jnp.float32)
                                 * recip)
            ctx_b = jnp.concatenate(ctx_parts, axis=1).astype(jnp.bfloat16)
            return jnp.dot(ctx_b, wo, preferred_element_type=jnp.float32)

        def exchange(s, p):
            rdma = pltpu.make_async_remote_copy(
                src_ref=send_ref.at[s, p],
                dst_ref=recv_ref.at[s, p],
                send_sem=send_sems.at[s, p],
                recv_sem=recv_sems.at[s, p],
                device_id=(jnp.bitwise_xor(my_pos, XORS[s][p]),),
                device_id_type=pl.DeviceIdType.MESH,
            )
            rdma.start()
            return rdma

        rdmas = {}
        for s, (b, r) in enumerate(STREAMS):
            part = compute_partial(b, r)
            out_ref[b, r * RH:(r + 1) * RH, :] = part
            send_ref[s, 0, :, :] = part.astype(jnp.bfloat16)
            rdmas[(s, 0)] = exchange(s, 0)

        for p in range(3):
            for s in range(4):
                b, r = STREAMS[s]
                rows = slice(r * RH, (r + 1) * RH)
                rdmas[(s, p)].wait()
                out_ref[b, rows, :] += recv_ref[s, p].astype(jnp.float32)
                if p < 2:
                    send_ref[s, p + 1, :, :] = out_ref[b, rows, :].astype(
                        jnp.bfloat16)
                    rdmas[(s, p + 1)] = exchange(s, p + 1)

    return pl.pallas_call(
        body,
        out_shape=jax.ShapeDtypeStruct((B, SQ, DM), jnp.float32),
        in_specs=[pl.BlockSpec(memory_space=pltpu.VMEM)] * 5,
        out_specs=pl.BlockSpec(memory_space=pltpu.VMEM),
        scratch_shapes=[
            pltpu.VMEM((4, 3, RH, DM), jnp.bfloat16),
            pltpu.VMEM((4, 3, RH, DM), jnp.bfloat16),
            pltpu.SemaphoreType.DMA((4, 3)),
            pltpu.SemaphoreType.DMA((4, 3)),
        ],
        compiler_params=pltpu.CompilerParams(collective_id=0),
    )(x, Wq_loc, K2, V2, Wo_loc)


# device time: 21638 ns/iter; 1.0023x vs baseline; 1.0023x over previous
import jax
import jax.numpy as jnp
from jax import lax
from jax.experimental import pallas as pl
from jax.experimental.pallas import tpu as pltpu

N_DEV = 8
B, SQ, SKV = 2, 256, 256
HQ_TOT, DH = 32, 64
H_LOC = HQ_TOT // N_DEV
CHUNK = H_LOC * DH
DM = 512

ROWS = ((0, 176), (176, 344), (344, 512))
RMAX = 176
XORS = ((1, 3, 4), (3, 4, 1), (4, 1, 3))


def kernel(x, Wq, K_ext, V_ext, Wo):
    my = lax.axis_index("i")
    Wq_loc = lax.dynamic_slice_in_dim(Wq, my * CHUNK, CHUNK, axis=1)
    Wo_loc = lax.dynamic_slice_in_dim(Wo, my * CHUNK, CHUNK, axis=0)
    K2 = K_ext.reshape(B, SKV, CHUNK)
    V2 = V_ext.reshape(B, SKV, CHUNK)

    def body(x_ref, wq_ref, k_ref, v_ref, wo_ref, out_ref,
             send_ref, recv_ref, send_sems, recv_sems):
        my_pos = lax.axis_index("i")

        barrier_sem = pltpu.get_barrier_semaphore()
        for d in (1, 3, 4):
            pl.semaphore_signal(barrier_sem, inc=1,
                                device_id=(jnp.bitwise_xor(my_pos, d),),
                                device_id_type=pl.DeviceIdType.MESH)

        qi = lax.broadcasted_iota(jnp.int32, (SQ, SKV), 0)
        ki = lax.broadcasted_iota(jnp.int32, (SQ, SKV), 1)
        mask = (jnp.abs(qi - ki) <= 128) | (ki < 32) | (qi < 32)

        wq = (wq_ref[...] * 0.125).astype(jnp.bfloat16)
        wo = wo_ref[...].astype(jnp.bfloat16)

        def compute_partial(b):
            xb = x_ref[b].astype(jnp.bfloat16)
            qb = jnp.dot(xb, wq, preferred_element_type=jnp.float32)
            ctx_parts = []
            for h in range(H_LOC):
                qh = qb[:, h * DH:(h + 1) * DH].astype(jnp.bfloat16)
                kh = k_ref[b, :, h * DH:(h + 1) * DH].astype(jnp.bfloat16)
                vh = v_ref[b, :, h * DH:(h + 1) * DH].astype(jnp.bfloat16)
                s = lax.dot_general(qh, kh, (((1,), (1,)), ((), ())),
                                    preferred_element_type=jnp.float32)
                w = jnp.where(mask, jnp.exp(s), 0.0)
                recip = 1.0 / jnp.sum(w, axis=-1, keepdims=True)
                ctx_parts.append(jnp.dot(w.astype(jnp.bfloat16), vh,
                                         preferred_element_type=jnp.float32)
                                 * recip)
            ctx_b = jnp.concatenate(ctx_parts, axis=1).astype(jnp.bfloat16)
            return jnp.dot(ctx_b, wo,
                           preferred_element_type=jnp.float32).astype(jnp.bfloat16)

        def exchange(s, p):
            n = ROWS[s][1] - ROWS[s][0]
            rdma = pltpu.make_async_remote_copy(
                src_ref=send_ref.at[s, p, pl.ds(0, n)],
                dst_ref=recv_ref.at[s, p, pl.ds(0, n)],
                send_sem=send_sems.at[s, p],
                recv_sem=recv_sems.at[s, p],
                device_id=(jnp.bitwise_xor(my_pos, XORS[s][p]),),
                device_id_type=pl.DeviceIdType.MESH,
            )
            rdma.start()
            return rdma

        p0 = compute_partial(0)
        send_ref[0, 0, 0:176, :] = p0[0:176, :]
        send_ref[1, 0, 0:80, :] = p0[176:256, :]
        pl.semaphore_wait(barrier_sem, 3)
        rdmas = {(0, 0): exchange(0, 0)}

        p1 = compute_partial(1)
        send_ref[1, 0, 80:168, :] = p1[0:88, :]
        send_ref[2, 0, 0:168, :] = p1[88:256, :]
        rdmas[(1, 0)] = exchange(1, 0)
        rdmas[(2, 0)] = exchange(2, 0)

        for p in range(3):
            for s in range(3):
                n = ROWS[s][1] - ROWS[s][0]
                rdmas[(s, p)].wait()
                if p < 2:
                    send_ref[s, p + 1, 0:n, :] = (
                        send_ref[s, p, 0:n, :] + recv_ref[s, p, 0:n, :])
                    rdmas[(s, p + 1)] = exchange(s, p + 1)
                else:
                    total = (send_ref[s, 2, 0:n, :].astype(jnp.float32)
                             + recv_ref[s, 2, 0:n, :].astype(jnp.float32))
                    if s == 0:
                        out_ref[0, 0:176, :] = total
                    elif s == 1:
                        out_ref[0, 176:256, :] = total[0:80, :]
                        out_ref[1, 0:88, :] = total[80:168, :]
                    else:
                        out_ref[1, 88:256, :] = total

    return pl.pallas_call(
        body,
        out_shape=jax.ShapeDtypeStruct((B, SQ, DM), jnp.float32),
        in_specs=[pl.BlockSpec(memory_space=pltpu.VMEM)] * 5,
        out_specs=pl.BlockSpec(memory_space=pltpu.VMEM),
        scratch_shapes=[
            pltpu.VMEM((3, 3, RMAX, DM), jnp.bfloat16),
            pltpu.VMEM((3, 3, RMAX, DM), jnp.bfloat16),
            pltpu.SemaphoreType.DMA((3, 3)),
            pltpu.SemaphoreType.DMA((3, 3)),
        ],
        compiler_params=pltpu.CompilerParams(collective_id=0),
    )(x, Wq_loc, K2, V2, Wo_loc)
